# Initial kernel scaffold; baseline (speedup 1.0000x reference)
import jax
import jax.numpy as jnp
from jax.experimental import pallas as pl


def kernel(x, edge_index, W1, att_src1, att_dst1, b1, W2, att_src2, att_dst2, b2):
    raise NotImplementedError("write your pallas kernel here")


# speedup vs baseline: 72.1044x; 72.1044x over previous
import functools

import jax
import jax.numpy as jnp
from jax import lax
from jax.experimental import pallas as pl
from jax.experimental.pallas import tpu as pltpu
from jax.experimental.pallas import tpu_sc as plsc

N = 10000
NP = 10240
E = 320000
EP = 327680
D_IN = 128
D_HID = 16
D_OUT = 2
NTILES = 32
EBLK = 2048
F32 = jnp.float32


def _tc1_body(x_ref, w_ref, asv_ref, adv_ref, h_ref, as_ref, ad_ref):
    h = jnp.dot(x_ref[...], w_ref[...], preferred_element_type=F32)
    h_ref[...] = h
    as_ref[...] = jnp.sum(h * asv_ref[...], axis=1, keepdims=True)
    ad_ref[...] = jnp.sum(h * adv_ref[...], axis=1, keepdims=True)


def _tc1(xp, W1, asv, adv):
    blk = 2560
    grid = NP // blk
    return pl.pallas_call(
        _tc1_body,
        grid=(grid,),
        in_specs=[
            pl.BlockSpec((blk, D_IN), lambda i: (i, 0)),
            pl.BlockSpec((D_IN, D_HID), lambda i: (0, 0)),
            pl.BlockSpec((1, D_HID), lambda i: (0, 0)),
            pl.BlockSpec((1, D_HID), lambda i: (0, 0)),
        ],
        out_specs=[
            pl.BlockSpec((blk, D_HID), lambda i: (i, 0)),
            pl.BlockSpec((blk, 1), lambda i: (i, 0)),
            pl.BlockSpec((blk, 1), lambda i: (i, 0)),
        ],
        out_shape=[
            jax.ShapeDtypeStruct((NP, D_HID), F32),
            jax.ShapeDtypeStruct((NP, 1), F32),
            jax.ShapeDtypeStruct((NP, 1), F32),
        ],
    )(xp, W1, asv, adv)


def _make_sc_edge_pass(d_cols, n_cgroups):
    cpt = d_cols // n_cgroups
    n_eg = NTILES // n_cgroups
    chunk = EP // n_eg
    n_blk = chunk // EBLK

    mesh = plsc.VectorSubcoreMesh(core_axis_name="c", subcore_axis_name="s")

    @functools.partial(
        pl.kernel,
        out_type=[
            jax.ShapeDtypeStruct((NTILES, cpt, NP), F32),
            jax.ShapeDtypeStruct((NTILES, NP), F32),
        ],
        mesh=mesh,
        compiler_params=pltpu.CompilerParams(needs_layout_passes=False),
        scratch_types=[
            pltpu.VMEM((NP,), F32),
            pltpu.VMEM((NP,), F32),
            pltpu.VMEM((cpt, NP), F32),
            pltpu.VMEM((cpt, NP), F32),
            pltpu.VMEM((NP,), F32),
            pltpu.VMEM((EBLK,), jnp.int32),
            pltpu.VMEM((EBLK,), jnp.int32),
        ],
    )
    def edge_pass(src_hbm, dst_hbm, ht_hbm, as_hbm, ad_hbm,
                  num_out, den_out, as_tab, ad_tab, hc, nacc, dacc,
                  sbuf, dbuf):
        cid = lax.axis_index("c")
        sid = lax.axis_index("s")
        wid = sid * 2 + cid
        eg = wid // n_cgroups
        cg = wid % n_cgroups

        pltpu.sync_copy(as_hbm, as_tab)
        pltpu.sync_copy(ad_hbm, ad_tab)
        pltpu.sync_copy(ht_hbm.at[pl.ds(cg * cpt, cpt)], hc)

        zero16 = jnp.zeros((16,), F32)

        def zbody(i, _):
            dacc[pl.ds(i * 16, 16)] = zero16
            for j in range(cpt):
                nacc[j, pl.ds(i * 16, 16)] = zero16
            return ()

        lax.fori_loop(0, NP // 16, zbody, ())

        base = eg * chunk

        def blk_body(b, _):
            off = base + b * EBLK
            pltpu.sync_copy(src_hbm.at[pl.ds(off, EBLK)], sbuf)
            pltpu.sync_copy(dst_hbm.at[pl.ds(off, EBLK)], dbuf)

            def ebody(i, _):
                s16 = sbuf[pl.ds(i * 16, 16)]
                d16 = dbuf[pl.ds(i * 16, 16)]
                av = plsc.load_gather(as_tab, [s16])
                bv = plsc.load_gather(ad_tab, [d16])
                e = av + bv
                e = jnp.where(e > 0, e, 0.2 * e)
                ex = jnp.exp(e)
                plsc.addupdate_scatter(dacc, [d16], ex)
                for j in range(cpt):
                    jv = jnp.full((16,), j, jnp.int32)
                    hv = plsc.load_gather(hc, [jv, s16])
                    plsc.addupdate_scatter(nacc, [jv, d16], hv * ex)
                return ()

            lax.fori_loop(0, EBLK // 16, ebody, ())
            return ()

        lax.fori_loop(0, n_blk, blk_body, ())

        pltpu.sync_copy(nacc, num_out.at[wid])
        pltpu.sync_copy(dacc, den_out.at[wid])

    return edge_pass


_sc_pass1 = _make_sc_edge_pass(D_HID, 4)
_sc_pass2 = _make_sc_edge_pass(D_OUT, 1)


def _tc2_body(numP_ref, denP_ref, h1t_ref, as_ref, ad_ref, b1_ref, w2_ref,
              asv2_ref, adv2_ref, h2t_ref, as2_ref, ad2_ref):
    rows = []
    for c in range(D_HID):
        cg, j = divmod(c, 4)
        v = numP_ref[cg, j:j + 1, :]
        for eg in range(1, 8):
            v = v + numP_ref[eg * 4 + cg, j:j + 1, :]
        rows.append(v)
    num = jnp.concatenate(rows, axis=0)
    den = denP_ref[0:1, :]
    for eg in range(1, 8):
        den = den + denP_ref[eg * 4:eg * 4 + 1, :]

    a = as_ref[...] + ad_ref[...]
    ex_self = jnp.exp(jnp.where(a > 0, a, 0.2 * a))
    h1t = h1t_ref[...]
    out1 = (num + ex_self * h1t) / (den + ex_self + 1e-16) + b1_ref[...]
    z = jnp.maximum(out1, 0.0)

    h2_rows = []
    for c in range(D_OUT):
        h2_rows.append(jnp.sum(z * w2_ref[:, c:c + 1], axis=0, keepdims=True))
    h2t = jnp.concatenate(h2_rows, axis=0)
    h2t_ref[...] = h2t
    as2_ref[...] = jnp.sum(h2t * asv2_ref[...], axis=0, keepdims=True)
    ad2_ref[...] = jnp.sum(h2t * adv2_ref[...], axis=0, keepdims=True)


def _tc2(numP, denP, h1t, a_s, a_d, b1, W2, asv2, adv2):
    return pl.pallas_call(
        _tc2_body,
        out_shape=[
            jax.ShapeDtypeStruct((D_OUT, NP), F32),
            jax.ShapeDtypeStruct((1, NP), F32),
            jax.ShapeDtypeStruct((1, NP), F32),
        ],
    )(numP, denP, h1t, a_s, a_d, b1, W2, asv2, adv2)


def _tc3_body(numP_ref, denP_ref, h2t_ref, as_ref, ad_ref, b2_ref, out_ref):
    rows = []
    for c in range(D_OUT):
        v = numP_ref[0, c:c + 1, :]
        for w in range(1, NTILES):
            v = v + numP_ref[w, c:c + 1, :]
        rows.append(v)
    num = jnp.concatenate(rows, axis=0)
    den = denP_ref[0:1, :]
    for w in range(1, NTILES):
        den = den + denP_ref[w:w + 1, :]

    a = as_ref[...] + ad_ref[...]
    ex_self = jnp.exp(jnp.where(a > 0, a, 0.2 * a))
    out2 = (num + ex_self * h2t_ref[...]) / (den + ex_self + 1e-16) + b2_ref[...]
    m = jnp.max(out2, axis=0, keepdims=True)
    lse = m + jnp.log(jnp.sum(jnp.exp(out2 - m), axis=0, keepdims=True))
    out_ref[...] = out2 - lse


def _tc3(numP, denP, h2t, as2, ad2, b2):
    return pl.pallas_call(
        _tc3_body,
        out_shape=jax.ShapeDtypeStruct((D_OUT, NP), F32),
    )(numP, denP, h2t, as2, ad2, b2)


def kernel(x, edge_index, W1, att_src1, att_dst1, b1, W2, att_src2, att_dst2,
           b2):
    pad = jnp.full((2, EP - E), N, jnp.int32)
    ei = jnp.concatenate([edge_index.astype(jnp.int32), pad], axis=1)
    src, dst = ei[0], ei[1]

    xp = jnp.concatenate([x, jnp.zeros((NP - N, D_IN), F32)], axis=0)
    h1, a_s1, a_d1 = _tc1(xp, W1, att_src1.reshape(1, D_HID),
                          att_dst1.reshape(1, D_HID))
    h1t = h1.T
    as1_row = a_s1.reshape(1, NP)
    ad1_row = a_d1.reshape(1, NP)

    numP1, denP1 = _sc_pass1(src, dst, h1t, a_s1.reshape(NP),
                             a_d1.reshape(NP))

    h2t, as2_row, ad2_row = _tc2(numP1, denP1, h1t, as1_row, ad1_row,
                                 b1.reshape(D_HID, 1), W2,
                                 att_src2.reshape(D_OUT, 1),
                                 att_dst2.reshape(D_OUT, 1))

    numP2, denP2 = _sc_pass2(src, dst, h2t, as2_row.reshape(NP),
                             ad2_row.reshape(NP))

    outT = _tc3(numP2, denP2, h2t, as2_row, ad2_row, b2.reshape(D_OUT, 1))
    return outT.T[:N]


# speedup vs baseline: 120.1451x; 1.6663x over previous
import functools

import jax
import jax.numpy as jnp
from jax import lax
from jax.experimental import pallas as pl
from jax.experimental.pallas import tpu as pltpu
from jax.experimental.pallas import tpu_sc as plsc

N = 10000
NP = 10240
E = 320000
EP = 327680
D_IN = 128
D_HID = 16
D_OUT = 2
NTILES = 32
EBLK = 2048
F32 = jnp.float32


def _tc1_body(x_ref, w_ref, asv_ref, adv_ref, h_ref, as_ref, ad_ref):
    h = jnp.dot(x_ref[...], w_ref[...], preferred_element_type=F32)
    h_ref[...] = h
    as_ref[...] = jnp.sum(h * asv_ref[...], axis=1, keepdims=True)
    ad_ref[...] = jnp.sum(h * adv_ref[...], axis=1, keepdims=True)


def _tc1(xp, W1, asv, adv):
    blk = 2560
    grid = NP // blk
    return pl.pallas_call(
        _tc1_body,
        grid=(grid,),
        in_specs=[
            pl.BlockSpec((blk, D_IN), lambda i: (i, 0)),
            pl.BlockSpec((D_IN, D_HID), lambda i: (0, 0)),
            pl.BlockSpec((1, D_HID), lambda i: (0, 0)),
            pl.BlockSpec((1, D_HID), lambda i: (0, 0)),
        ],
        out_specs=[
            pl.BlockSpec((blk, D_HID), lambda i: (i, 0)),
            pl.BlockSpec((blk, 1), lambda i: (i, 0)),
            pl.BlockSpec((blk, 1), lambda i: (i, 0)),
        ],
        out_shape=[
            jax.ShapeDtypeStruct((NP, D_HID), F32),
            jax.ShapeDtypeStruct((NP, 1), F32),
            jax.ShapeDtypeStruct((NP, 1), F32),
        ],
    )(xp, W1, asv, adv)


def _make_sc_edge_pass(d_cols, n_cgroups, eblk):
    cpt = d_cols // n_cgroups
    n_eg = NTILES // n_cgroups
    chunk = EP // n_eg
    n_blk = chunk // eblk
    assert n_blk % 2 == 0

    mesh = plsc.VectorSubcoreMesh(core_axis_name="c", subcore_axis_name="s")

    @functools.partial(
        pl.kernel,
        out_type=[
            jax.ShapeDtypeStruct((NTILES, cpt, NP), F32),
            jax.ShapeDtypeStruct((NTILES, NP), F32),
        ],
        mesh=mesh,
        compiler_params=pltpu.CompilerParams(needs_layout_passes=False),
        scratch_types=[
            pltpu.VMEM((NP,), F32),
            pltpu.VMEM((NP,), F32),
            pltpu.VMEM((cpt, NP), F32),
            pltpu.VMEM((cpt, NP), F32),
            pltpu.VMEM((NP,), F32),
            pltpu.VMEM((2, eblk), jnp.int32),
            pltpu.VMEM((2, eblk), jnp.int32),
            pltpu.SemaphoreType.DMA,
            pltpu.SemaphoreType.DMA((2,)),
            pltpu.SemaphoreType.DMA((2,)),
        ],
    )
    def edge_pass(src_hbm, dst_hbm, ht_hbm, as_hbm, ad_hbm,
                  num_out, den_out, as_tab, ad_tab, hc, nacc, dacc,
                  sbuf, dbuf, sem_t, sem_s, sem_d):
        cid = lax.axis_index("c")
        sid = lax.axis_index("s")
        wid = sid * 2 + cid
        eg = wid // n_cgroups
        cg = wid % n_cgroups
        base = eg * chunk

        t1 = pltpu.async_copy(as_hbm, as_tab, sem_t)
        t2 = pltpu.async_copy(ad_hbm, ad_tab, sem_t)
        t3 = pltpu.async_copy(ht_hbm.at[pl.ds(cg * cpt, cpt)], hc, sem_t)
        for k in (0, 1):
            pltpu.async_copy(src_hbm.at[pl.ds(base + k * eblk, eblk)],
                             sbuf.at[k], sem_s.at[k])
            pltpu.async_copy(dst_hbm.at[pl.ds(base + k * eblk, eblk)],
                             dbuf.at[k], sem_d.at[k])

        zero16 = jnp.zeros((16,), F32)

        @plsc.parallel_loop(0, NP // 16, unroll=4)
        def _(i):
            dacc[pl.ds(i * 16, 16)] = zero16
            for j in range(cpt):
                nacc[j, pl.ds(i * 16, 16)] = zero16

        t1.wait()
        t2.wait()
        t3.wait()

        def blk_body(bb, _):
            for k in (0, 1):
                b = bb * 2 + k
                pltpu.make_async_copy(src_hbm.at[pl.ds(base, eblk)],
                                      sbuf.at[k], sem_s.at[k]).wait()
                pltpu.make_async_copy(dst_hbm.at[pl.ds(base, eblk)],
                                      dbuf.at[k], sem_d.at[k]).wait()

                @plsc.parallel_loop(0, eblk // 16, unroll=4)
                def _(i):
                    s16 = sbuf[k, pl.ds(i * 16, 16)]
                    d16 = dbuf[k, pl.ds(i * 16, 16)]
                    av = plsc.load_gather(as_tab, [s16])
                    bv = plsc.load_gather(ad_tab, [d16])
                    e = av + bv
                    e = jnp.where(e > 0, e, 0.2 * e)
                    ex = jnp.exp(e)
                    plsc.addupdate_scatter(dacc, [d16], ex)
                    for j in range(cpt):
                        jv = jnp.full((16,), j, jnp.int32)
                        hv = plsc.load_gather(hc, [jv, s16])
                        plsc.addupdate_scatter(nacc, [jv, d16], hv * ex)

                @pl.when(b + 2 < n_blk)
                def _():
                    off = base + (b + 2) * eblk
                    pltpu.async_copy(src_hbm.at[pl.ds(off, eblk)],
                                     sbuf.at[k], sem_s.at[k])
                    pltpu.async_copy(dst_hbm.at[pl.ds(off, eblk)],
                                     dbuf.at[k], sem_d.at[k])
            return ()

        lax.fori_loop(0, n_blk // 2, blk_body, ())

        pltpu.sync_copy(nacc, num_out.at[wid])
        pltpu.sync_copy(dacc, den_out.at[wid])

    return edge_pass


_sc_pass1 = _make_sc_edge_pass(D_HID, 4, 2048)
_sc_pass2 = _make_sc_edge_pass(D_OUT, 1, 1024)


def _tc2_body(numP_ref, denP_ref, h1t_ref, as_ref, ad_ref, b1_ref, w2_ref,
              asv2_ref, adv2_ref, h2t_ref, as2_ref, ad2_ref):
    rows = []
    for c in range(D_HID):
        cg, j = divmod(c, 4)
        v = numP_ref[cg, j:j + 1, :]
        for eg in range(1, 8):
            v = v + numP_ref[eg * 4 + cg, j:j + 1, :]
        rows.append(v)
    num = jnp.concatenate(rows, axis=0)
    den = denP_ref[0:1, :]
    for eg in range(1, 8):
        den = den + denP_ref[eg * 4:eg * 4 + 1, :]

    a = as_ref[...] + ad_ref[...]
    ex_self = jnp.exp(jnp.where(a > 0, a, 0.2 * a))
    h1t = h1t_ref[...]
    out1 = (num + ex_self * h1t) / (den + ex_self + 1e-16) + b1_ref[...]
    z = jnp.maximum(out1, 0.0)

    h2_rows = []
    for c in range(D_OUT):
        h2_rows.append(jnp.sum(z * w2_ref[:, c:c + 1], axis=0, keepdims=True))
    h2t = jnp.concatenate(h2_rows, axis=0)
    h2t_ref[...] = h2t
    as2_ref[...] = jnp.sum(h2t * asv2_ref[...], axis=0, keepdims=True)
    ad2_ref[...] = jnp.sum(h2t * adv2_ref[...], axis=0, keepdims=True)


def _tc2(numP, denP, h1t, a_s, a_d, b1, W2, asv2, adv2):
    return pl.pallas_call(
        _tc2_body,
        out_shape=[
            jax.ShapeDtypeStruct((D_OUT, NP), F32),
            jax.ShapeDtypeStruct((1, NP), F32),
            jax.ShapeDtypeStruct((1, NP), F32),
        ],
    )(numP, denP, h1t, a_s, a_d, b1, W2, asv2, adv2)


def _tc3_body(numP_ref, denP_ref, h2t_ref, as_ref, ad_ref, b2_ref, out_ref):
    rows = []
    for c in range(D_OUT):
        v = numP_ref[0, c:c + 1, :]
        for w in range(1, NTILES):
            v = v + numP_ref[w, c:c + 1, :]
        rows.append(v)
    num = jnp.concatenate(rows, axis=0)
    den = denP_ref[0:1, :]
    for w in range(1, NTILES):
        den = den + denP_ref[w:w + 1, :]

    a = as_ref[...] + ad_ref[...]
    ex_self = jnp.exp(jnp.where(a > 0, a, 0.2 * a))
    out2 = (num + ex_self * h2t_ref[...]) / (den + ex_self + 1e-16) + b2_ref[...]
    m = jnp.max(out2, axis=0, keepdims=True)
    lse = m + jnp.log(jnp.sum(jnp.exp(out2 - m), axis=0, keepdims=True))
    out_ref[...] = out2 - lse


def _tc3(numP, denP, h2t, as2, ad2, b2):
    return pl.pallas_call(
        _tc3_body,
        out_shape=jax.ShapeDtypeStruct((D_OUT, NP), F32),
    )(numP, denP, h2t, as2, ad2, b2)


def kernel(x, edge_index, W1, att_src1, att_dst1, b1, W2, att_src2, att_dst2,
           b2):
    pad = jnp.full((2, EP - E), N, jnp.int32)
    ei = jnp.concatenate([edge_index.astype(jnp.int32), pad], axis=1)
    src, dst = ei[0], ei[1]

    xp = jnp.concatenate([x, jnp.zeros((NP - N, D_IN), F32)], axis=0)
    h1, a_s1, a_d1 = _tc1(xp, W1, att_src1.reshape(1, D_HID),
                          att_dst1.reshape(1, D_HID))
    h1t = h1.T
    as1_row = a_s1.reshape(1, NP)
    ad1_row = a_d1.reshape(1, NP)

    numP1, denP1 = _sc_pass1(src, dst, h1t, a_s1.reshape(NP),
                             a_d1.reshape(NP))

    h2t, as2_row, ad2_row = _tc2(numP1, denP1, h1t, as1_row, ad1_row,
                                 b1.reshape(D_HID, 1), W2,
                                 att_src2.reshape(D_OUT, 1),
                                 att_dst2.reshape(D_OUT, 1))

    numP2, denP2 = _sc_pass2(src, dst, h2t, as2_row.reshape(NP),
                             ad2_row.reshape(NP))

    outT = _tc3(numP2, denP2, h2t, as2_row, ad2_row, b2.reshape(D_OUT, 1))
    return outT.T[:N]
